# Initial kernel scaffold; baseline (speedup 1.0000x reference)
import jax
import jax.numpy as jnp
from jax.experimental import pallas as pl


def kernel(inp, daytime, W_day, W_time):
    raise NotImplementedError("write your pallas kernel here")


# speedup vs baseline: 2.2328x; 2.2328x over previous
import jax
import jax.numpy as jnp
from jax import lax
from jax.experimental import pallas as pl

_BN = 1024


def _body(inp_ref, idx_ref, wd_ref, wt_ref, out_ref):
    d = idx_ref[:, 0]
    t = idx_ref[:, 1]
    oh_d = (d[:, None] == lax.broadcasted_iota(jnp.int32, (_BN, 8), 1)
            ).astype(jnp.float32)
    day = jnp.dot(oh_d, wd_ref[...], preferred_element_type=jnp.float32)
    oh_t = (t[:, None] == lax.broadcasted_iota(jnp.int32, (_BN, 288), 1)
            ).astype(jnp.float32)
    time = jnp.dot(oh_t, wt_ref[...], preferred_element_type=jnp.float32)
    out_ref[...] = jnp.concatenate([inp_ref[...], day, time], axis=-1)


def kernel(inp, daytime, W_day, W_time):
    B, T, C = inp.shape
    N = B * T
    inp2 = inp.reshape(N, C)
    idx2 = daytime.astype(jnp.int32).reshape(N, 2)
    wd = jnp.pad(W_day, ((0, 1), (0, 0)))
    Dd, Dt = W_day.shape[1], W_time.shape[1]
    out = pl.pallas_call(
        _body,
        grid=(N // _BN,),
        in_specs=[
            pl.BlockSpec((_BN, C), lambda i: (i, 0)),
            pl.BlockSpec((_BN, 2), lambda i: (i, 0)),
            pl.BlockSpec((8, Dd), lambda i: (0, 0)),
            pl.BlockSpec(W_time.shape, lambda i: (0, 0)),
        ],
        out_specs=pl.BlockSpec((_BN, C + Dd + Dt), lambda i: (i, 0)),
        out_shape=jax.ShapeDtypeStruct((N, C + Dd + Dt), jnp.float32),
    )(inp2, idx2, wd, W_time)
    return out.reshape(B, T, C + Dd + Dt)


# speedup vs baseline: 2.4561x; 1.1000x over previous
import jax
import jax.numpy as jnp
from jax import lax
from jax.experimental import pallas as pl

_BB = 8


def _body(inp_ref, idx_ref, wd_ref, wt_ref, out_ref):
    n = _BB * 50
    d = idx_ref[:, 0]
    t = idx_ref[:, 1]
    oh_d = (d[:, None] == lax.broadcasted_iota(jnp.int32, (n, 8), 1)
            ).astype(jnp.float32)
    day = jnp.dot(oh_d, wd_ref[...], preferred_element_type=jnp.float32)
    oh_t = (t[:, None] == lax.broadcasted_iota(jnp.int32, (n, 288), 1)
            ).astype(jnp.float32)
    time = jnp.dot(oh_t, wt_ref[...], preferred_element_type=jnp.float32)
    dt = jnp.concatenate([day, time], axis=-1)
    out_ref[:, :, :128] = inp_ref[...]
    out_ref[:, :, 128:] = dt.reshape(_BB, 50, 48)


def kernel(inp, daytime, W_day, W_time):
    B, T, C = inp.shape
    idx2 = daytime.astype(jnp.int32).reshape(B * T, 2)
    wd = jnp.pad(W_day, ((0, 1), (0, 0)))
    Dd, Dt = W_day.shape[1], W_time.shape[1]
    n = _BB * T
    return pl.pallas_call(
        _body,
        grid=(B // _BB,),
        in_specs=[
            pl.BlockSpec((_BB, T, C), lambda i: (i, 0, 0)),
            pl.BlockSpec((n, 2), lambda i: (i, 0)),
            pl.BlockSpec((8, Dd), lambda i: (0, 0)),
            pl.BlockSpec(W_time.shape, lambda i: (0, 0)),
        ],
        out_specs=pl.BlockSpec((_BB, T, C + Dd + Dt), lambda i: (i, 0, 0)),
        out_shape=jax.ShapeDtypeStruct((B, T, C + Dd + Dt), jnp.float32),
    )(inp, idx2, wd, W_time)


# speedup vs baseline: 4.4949x; 1.8301x over previous
import jax
import jax.numpy as jnp
from jax import lax
from jax.experimental import pallas as pl

_BB = 64


def _body(inp_ref, cid_ref, wd_ref, wt_ref, out_ref):
    n = _BB * 50
    c = cid_ref[0, 0, :].reshape(n, 1)
    d = c >> 9
    t = c & 511
    oh_d = (d == lax.broadcasted_iota(jnp.int32, (n, 8), 1)).astype(jnp.float32)
    day = jnp.dot(oh_d, wd_ref[...], preferred_element_type=jnp.float32)
    oh_t = (t == lax.broadcasted_iota(jnp.int32, (n, 288), 1)
            ).astype(jnp.float32)
    time = jnp.dot(oh_t, wt_ref[...], preferred_element_type=jnp.float32)
    dt = jnp.concatenate([day, time], axis=-1)
    out_ref[:, :, :128] = inp_ref[...]
    out_ref[:, :, 128:] = dt.reshape(_BB, 50, 48)


def kernel(inp, daytime, W_day, W_time):
    B, T, C = inp.shape
    n = _BB * T
    nb = B // _BB
    dt32 = daytime.astype(jnp.int32)
    cid = ((dt32[:, :, 0] << 9) | dt32[:, :, 1]).reshape(nb, 1, n)
    wd = jnp.pad(W_day, ((0, 1), (0, 0)))
    Dd, Dt = W_day.shape[1], W_time.shape[1]
    return pl.pallas_call(
        _body,
        grid=(nb,),
        in_specs=[
            pl.BlockSpec((_BB, T, C), lambda i: (i, 0, 0)),
            pl.BlockSpec((1, 1, n), lambda i: (i, 0, 0)),
            pl.BlockSpec((8, Dd), lambda i: (0, 0)),
            pl.BlockSpec(W_time.shape, lambda i: (0, 0)),
        ],
        out_specs=pl.BlockSpec((_BB, T, C + Dd + Dt), lambda i: (i, 0, 0)),
        out_shape=jax.ShapeDtypeStruct((B, T, C + Dd + Dt), jnp.float32),
    )(inp, cid, wd, W_time)


# speedup vs baseline: 4.6996x; 1.0456x over previous
import jax
import jax.numpy as jnp
from jax import lax
from jax.experimental import pallas as pl

_BB = 128


def _body(inp_ref, cid_ref, wd_ref, wt_ref, out_ref):
    n = _BB * 50
    c = cid_ref[0, 0, :].reshape(n, 1)
    d = c >> 9
    t = c & 511
    oh_d = (d == lax.broadcasted_iota(jnp.int32, (n, 8), 1)).astype(jnp.float32)
    day = jnp.dot(oh_d, wd_ref[...], preferred_element_type=jnp.float32)
    oh_t = (t == lax.broadcasted_iota(jnp.int32, (n, 288), 1)
            ).astype(jnp.float32)
    time = jnp.dot(oh_t, wt_ref[...], preferred_element_type=jnp.float32)
    dt = jnp.concatenate([day, time], axis=-1)
    out_ref[:, :, :128] = inp_ref[...]
    out_ref[:, :, 128:] = dt.reshape(_BB, 50, 48)


def kernel(inp, daytime, W_day, W_time):
    B, T, C = inp.shape
    n = _BB * T
    nb = B // _BB
    dt32 = daytime.astype(jnp.int32)
    cid = ((dt32[:, :, 0] << 9) | dt32[:, :, 1]).reshape(nb, 1, n)
    wd = jnp.pad(W_day, ((0, 1), (0, 0)))
    Dd, Dt = W_day.shape[1], W_time.shape[1]
    return pl.pallas_call(
        _body,
        grid=(nb,),
        in_specs=[
            pl.BlockSpec((_BB, T, C), lambda i: (i, 0, 0)),
            pl.BlockSpec((1, 1, n), lambda i: (i, 0, 0)),
            pl.BlockSpec((8, Dd), lambda i: (0, 0)),
            pl.BlockSpec(W_time.shape, lambda i: (0, 0)),
        ],
        out_specs=pl.BlockSpec((_BB, T, C + Dd + Dt), lambda i: (i, 0, 0)),
        out_shape=jax.ShapeDtypeStruct((B, T, C + Dd + Dt), jnp.float32),
    )(inp, cid, wd, W_time)


# speedup vs baseline: 4.7899x; 1.0192x over previous
import jax
import jax.numpy as jnp
from jax import lax
from jax.experimental import pallas as pl

_BB = 256


def _body(inp_ref, cid_ref, wd_ref, wt_ref, out_ref):
    n = _BB * 50
    c = cid_ref[0, 0, :].reshape(n, 1)
    d = c >> 9
    t = c & 511
    oh_d = (d == lax.broadcasted_iota(jnp.int32, (n, 8), 1)).astype(jnp.float32)
    day = jnp.dot(oh_d, wd_ref[...], preferred_element_type=jnp.float32)
    oh_t = (t == lax.broadcasted_iota(jnp.int32, (n, 288), 1)
            ).astype(jnp.float32)
    time = jnp.dot(oh_t, wt_ref[...], preferred_element_type=jnp.float32)
    dt = jnp.concatenate([day, time], axis=-1)
    out_ref[:, :, :128] = inp_ref[...]
    out_ref[:, :, 128:] = dt.reshape(_BB, 50, 48)


def kernel(inp, daytime, W_day, W_time):
    B, T, C = inp.shape
    n = _BB * T
    nb = B // _BB
    dt32 = daytime.astype(jnp.int32)
    cid = ((dt32[:, :, 0] << 9) | dt32[:, :, 1]).reshape(nb, 1, n)
    wd = jnp.pad(W_day, ((0, 1), (0, 0)))
    Dd, Dt = W_day.shape[1], W_time.shape[1]
    return pl.pallas_call(
        _body,
        grid=(nb,),
        in_specs=[
            pl.BlockSpec((_BB, T, C), lambda i: (i, 0, 0)),
            pl.BlockSpec((1, 1, n), lambda i: (i, 0, 0)),
            pl.BlockSpec((8, Dd), lambda i: (0, 0)),
            pl.BlockSpec(W_time.shape, lambda i: (0, 0)),
        ],
        out_specs=pl.BlockSpec((_BB, T, C + Dd + Dt), lambda i: (i, 0, 0)),
        out_shape=jax.ShapeDtypeStruct((B, T, C + Dd + Dt), jnp.float32),
    )(inp, cid, wd, W_time)
